# initial kernel scaffold (unmeasured)
import jax
import jax.numpy as jnp
from jax import lax
from jax.experimental import pallas as pl
from jax.experimental.pallas import tpu as pltpu


def kernel(partial, resid, gamma):
    m, d = resid.shape

    def body(partial_ref, resid_ref, gamma_ref, out_ref,
             comm_ref, send_sem, recv_sem):
        my_x = lax.axis_index("x")
        my_y = lax.axis_index("y")
        my_z = lax.axis_index("z")
        nbr = (my_x, 1 - my_y, my_z)

        barrier_sem = pltpu.get_barrier_semaphore()
        pl.semaphore_signal(barrier_sem, inc=1, device_id=nbr,
                            device_id_type=pl.DeviceIdType.MESH)
        pl.semaphore_wait(barrier_sem, 1)

        rdma = pltpu.make_async_remote_copy(
            src_ref=partial_ref,
            dst_ref=comm_ref,
            send_sem=send_sem,
            recv_sem=recv_sem,
            device_id=nbr,
            device_id_type=pl.DeviceIdType.MESH,
        )
        rdma.start()
        rdma.wait()

        y = partial_ref[0] + comm_ref[0] + resid_ref[...]
        rms = jnp.sqrt(jnp.mean(y * y, axis=-1, keepdims=True) + 1e-6)
        out_ref[...] = y / rms * gamma_ref[...]


    return pl.pallas_call(
        body,
        out_shape=jax.ShapeDtypeStruct((m, d), jnp.float32),
        in_specs=[
            pl.BlockSpec(memory_space=pltpu.VMEM),
            pl.BlockSpec(memory_space=pltpu.VMEM),
            pl.BlockSpec(memory_space=pltpu.VMEM),
        ],
        out_specs=pl.BlockSpec(memory_space=pltpu.VMEM),
        scratch_shapes=[
            pltpu.VMEM((1, m, d), jnp.float32),
            pltpu.SemaphoreType.DMA,
            pltpu.SemaphoreType.DMA,
        ],
        compiler_params=pltpu.CompilerParams(collective_id=0),
    )(partial, resid, gamma.reshape(1, d))


# baseline (device time: 217880 ns/iter reference)
import jax
import jax.numpy as jnp
from jax import lax
from jax.experimental import pallas as pl
from jax.experimental.pallas import tpu as pltpu


def kernel(partial, resid, gamma):
    m, d = resid.shape

    n_blk = 16
    rows = m // n_blk

    def body(partial_ref, resid_ref, gamma_ref, out_ref,
             send_sem, recv_sem):
        my_x = lax.axis_index("x")
        my_y = lax.axis_index("y")
        my_z = lax.axis_index("z")
        nbr = (my_x, 1 - my_y, my_z)

        barrier_sem = pltpu.get_barrier_semaphore()
        pl.semaphore_signal(barrier_sem, inc=1, device_id=nbr,
                            device_id_type=pl.DeviceIdType.MESH)
        pl.semaphore_wait(barrier_sem, 1)

        rdma = pltpu.make_async_remote_copy(
            src_ref=partial_ref.at[0],
            dst_ref=out_ref,
            send_sem=send_sem,
            recv_sem=recv_sem,
            device_id=nbr,
            device_id_type=pl.DeviceIdType.MESH,
        )
        rdma.start()
        rdma.wait()

        def blk(b, carry):
            sl = pl.ds(b * rows, rows)
            y = out_ref[sl, :] + partial_ref[0, sl, :] + resid_ref[sl, :]
            rms = jnp.sqrt(jnp.mean(y * y, axis=-1, keepdims=True) + 1e-6)
            out_ref[sl, :] = y / rms * gamma_ref[...]
            return carry

        lax.fori_loop(0, n_blk, blk, 0)


    return pl.pallas_call(
        body,
        out_shape=jax.ShapeDtypeStruct((m, d), jnp.float32),
        in_specs=[
            pl.BlockSpec(memory_space=pltpu.VMEM),
            pl.BlockSpec(memory_space=pltpu.VMEM),
            pl.BlockSpec(memory_space=pltpu.VMEM),
        ],
        out_specs=pl.BlockSpec(memory_space=pltpu.VMEM),
        scratch_shapes=[
            pltpu.SemaphoreType.DMA,
            pltpu.SemaphoreType.DMA,
        ],
        compiler_params=pltpu.CompilerParams(
            collective_id=0,
            vmem_limit_bytes=62 * 1024 * 1024,
        ),
    )(partial, resid, gamma.reshape(1, d))


# device time: 131428 ns/iter; 1.6578x vs baseline; 1.6578x over previous
import jax
import jax.numpy as jnp
from jax import lax
from jax.experimental import pallas as pl
from jax.experimental.pallas import tpu as pltpu

K = 16


def kernel(partial, resid, gamma):
    m, d = resid.shape
    half = m // 2
    rows_c = half // K

    def body(partial_ref, resid_ref, gamma_ref, out_ref,
             y_send, y_recv, x_send, x_recv):
        my_x = lax.axis_index("x")
        my_y = lax.axis_index("y")
        my_z = lax.axis_index("z")
        y_nbr = (my_x, 1 - my_y, my_z)
        x_nbr = (1 - my_x, my_y, my_z)

        own0 = my_x * half
        oth0 = (1 - my_x) * half

        barrier_sem = pltpu.get_barrier_semaphore()
        for nbr in (y_nbr, x_nbr):
            pl.semaphore_signal(barrier_sem, inc=1, device_id=nbr,
                                device_id_type=pl.DeviceIdType.MESH)
        pl.semaphore_wait(barrier_sem, 2)

        def rdma_y(c):
            sl = pl.ds(own0 + c * rows_c, rows_c)
            return pltpu.make_async_remote_copy(
                src_ref=partial_ref.at[sl, :],
                dst_ref=out_ref.at[sl, :],
                send_sem=y_send.at[c],
                recv_sem=y_recv.at[c],
                device_id=y_nbr,
                device_id_type=pl.DeviceIdType.MESH,
            )

        def rdma_x(c):
            sl = pl.ds(own0 + c * rows_c, rows_c)
            return pltpu.make_async_remote_copy(
                src_ref=out_ref.at[sl, :],
                dst_ref=out_ref.at[sl, :],
                send_sem=x_send.at[c],
                recv_sem=x_recv.at[c],
                device_id=x_nbr,
                device_id_type=pl.DeviceIdType.MESH,
            )

        def rdma_x_recv(c):
            sl = pl.ds(oth0 + c * rows_c, rows_c)
            return pltpu.make_async_remote_copy(
                src_ref=out_ref.at[sl, :],
                dst_ref=out_ref.at[sl, :],
                send_sem=x_send.at[c],
                recv_sem=x_recv.at[c],
                device_id=x_nbr,
                device_id_type=pl.DeviceIdType.MESH,
            )

        def norm_chunk(row0):
            sl = pl.ds(row0, rows_c)
            y = out_ref[sl, :] + resid_ref[sl, :]
            rms = jnp.sqrt(jnp.mean(y * y, axis=-1, keepdims=True) + 1e-6)
            out_ref[sl, :] = y / rms * gamma_ref[...]

        def issue_y(c, carry):
            rdma_y(c).start()
            return carry
        lax.fori_loop(0, K, issue_y, 0)

        def reduce_fwd(c, carry):
            rdma_y(c).wait()
            sl = pl.ds(own0 + c * rows_c, rows_c)
            out_ref[sl, :] = out_ref[sl, :] + partial_ref[sl, :]
            rdma_x(c).start()
            return carry
        lax.fori_loop(0, K, reduce_fwd, 0)

        def finalize(c, carry):
            rdma_x(c).wait_send()
            norm_chunk(own0 + c * rows_c)
            rdma_x_recv(c).wait_recv()
            norm_chunk(oth0 + c * rows_c)
            return carry
        lax.fori_loop(0, K, finalize, 0)

    return pl.pallas_call(
        body,
        out_shape=jax.ShapeDtypeStruct((m, d), jnp.float32),
        in_specs=[
            pl.BlockSpec(memory_space=pltpu.VMEM),
            pl.BlockSpec(memory_space=pltpu.VMEM),
            pl.BlockSpec(memory_space=pltpu.VMEM),
        ],
        out_specs=pl.BlockSpec(memory_space=pltpu.VMEM),
        scratch_shapes=[
            pltpu.SemaphoreType.DMA((K,)),
            pltpu.SemaphoreType.DMA((K,)),
            pltpu.SemaphoreType.DMA((K,)),
            pltpu.SemaphoreType.DMA((K,)),
        ],
        compiler_params=pltpu.CompilerParams(
            collective_id=0,
            vmem_limit_bytes=62 * 1024 * 1024,
        ),
    )(partial.reshape(m, d), resid, gamma.reshape(1, d))


# device time: 131346 ns/iter; 1.6588x vs baseline; 1.0006x over previous
import jax
import jax.numpy as jnp
from jax import lax
from jax.experimental import pallas as pl
from jax.experimental.pallas import tpu as pltpu

K = 16


def kernel(partial, resid, gamma):
    m, d = resid.shape
    half = m // 2
    rows_c = half // K

    def body(partial_ref, resid_ref, gamma_ref, out_ref,
             y_send, y_recv, x_send, x_recv):
        my_x = lax.axis_index("x")
        my_y = lax.axis_index("y")
        my_z = lax.axis_index("z")
        y_nbr = (my_x, 1 - my_y, my_z)
        x_nbr = (1 - my_x, my_y, my_z)

        own0 = my_x * half
        oth0 = (1 - my_x) * half

        barrier_sem = pltpu.get_barrier_semaphore()
        for nbr in (y_nbr, x_nbr):
            pl.semaphore_signal(barrier_sem, inc=1, device_id=nbr,
                                device_id_type=pl.DeviceIdType.MESH)
        pl.semaphore_wait(barrier_sem, 2)

        def rdma_y(c):
            sl = pl.ds(own0 + c * rows_c, rows_c)
            return pltpu.make_async_remote_copy(
                src_ref=partial_ref.at[sl, :],
                dst_ref=out_ref.at[sl, :],
                send_sem=y_send.at[c],
                recv_sem=y_recv.at[c],
                device_id=y_nbr,
                device_id_type=pl.DeviceIdType.MESH,
            )

        def rdma_x(c):
            sl = pl.ds(own0 + c * rows_c, rows_c)
            return pltpu.make_async_remote_copy(
                src_ref=out_ref.at[sl, :],
                dst_ref=out_ref.at[sl, :],
                send_sem=x_send.at[c],
                recv_sem=x_recv.at[c],
                device_id=x_nbr,
                device_id_type=pl.DeviceIdType.MESH,
            )

        def rdma_x_recv(c):
            sl = pl.ds(oth0 + c * rows_c, rows_c)
            return pltpu.make_async_remote_copy(
                src_ref=out_ref.at[sl, :],
                dst_ref=out_ref.at[sl, :],
                send_sem=x_send.at[c],
                recv_sem=x_recv.at[c],
                device_id=x_nbr,
                device_id_type=pl.DeviceIdType.MESH,
            )

        def norm_chunk(row0):
            sl = pl.ds(row0, rows_c)
            y = out_ref[sl, :] + resid_ref[sl, :]
            rms = jnp.sqrt(jnp.mean(y * y, axis=-1, keepdims=True) + 1e-6)
            out_ref[sl, :] = y / rms * gamma_ref[...]

        def issue_y(c, carry):
            rdma_y(c).start()
            return carry
        lax.fori_loop(0, K, issue_y, 0)

        LAG = 3

        def step(c, carry):
            @pl.when(c < K)
            def _():
                rdma_y(c).wait()
                sl = pl.ds(own0 + c * rows_c, rows_c)
                out_ref[sl, :] = out_ref[sl, :] + partial_ref[sl, :]
                rdma_x(c).start()

            @pl.when(c >= LAG)
            def _():
                s = c - LAG
                rdma_x(s).wait_send()
                norm_chunk(own0 + s * rows_c)
                rdma_x_recv(s).wait_recv()
                norm_chunk(oth0 + s * rows_c)

            return carry
        lax.fori_loop(0, K + LAG, step, 0)

    return pl.pallas_call(
        body,
        out_shape=jax.ShapeDtypeStruct((m, d), jnp.float32),
        in_specs=[
            pl.BlockSpec(memory_space=pltpu.VMEM),
            pl.BlockSpec(memory_space=pltpu.VMEM),
            pl.BlockSpec(memory_space=pltpu.VMEM),
        ],
        out_specs=pl.BlockSpec(memory_space=pltpu.VMEM),
        scratch_shapes=[
            pltpu.SemaphoreType.DMA((K,)),
            pltpu.SemaphoreType.DMA((K,)),
            pltpu.SemaphoreType.DMA((K,)),
            pltpu.SemaphoreType.DMA((K,)),
        ],
        compiler_params=pltpu.CompilerParams(
            collective_id=0,
            vmem_limit_bytes=62 * 1024 * 1024,
        ),
    )(partial.reshape(m, d), resid, gamma.reshape(1, d))
